# Initial kernel scaffold; baseline (speedup 1.0000x reference)
import jax
import jax.numpy as jnp
from jax.experimental import pallas as pl


def kernel(q, k, v, proj):
    raise NotImplementedError("write your pallas kernel here")


# speedup vs baseline: 1.1132x; 1.1132x over previous
import functools

import jax
import jax.numpy as jnp
from jax.experimental import pallas as pl

N_HASHES = 8
BUCKET = 64


def _hash_body(q_ref, k_ref, p_ref, bq_ref, bk_ref):
    pq = jnp.dot(q_ref[0], p_ref[...], preferred_element_type=jnp.float32)
    pk = jnp.dot(k_ref[0], p_ref[...], preferred_element_type=jnp.float32)
    big = jnp.int32(1 << 30)
    for src, dst in ((pq, bq_ref), (pk, bk_ref)):
        for r in range(N_HASHES):
            pr = src[:, r * BUCKET:(r + 1) * BUCKET]
            iota = jax.lax.broadcasted_iota(jnp.int32, pr.shape, 1)
            mx = jnp.max(pr, axis=1)
            mn = jnp.min(pr, axis=1)
            amx = jnp.min(jnp.where(pr == mx[:, None], iota, big), axis=1)
            amn = jnp.min(jnp.where(pr == mn[:, None], iota, big), axis=1)
            dst[0, r, :] = jnp.where(mx >= -mn, amx, amn + BUCKET)


def _hash_pallas(q, k, proj2):
    b, t, d = q.shape
    tt = 512
    nc = proj2.shape[1]
    grid = (b, t // tt)
    return pl.pallas_call(
        _hash_body,
        grid=grid,
        in_specs=[
            pl.BlockSpec((1, tt, d), lambda bi, ti: (bi, ti, 0)),
            pl.BlockSpec((1, tt, d), lambda bi, ti: (bi, ti, 0)),
            pl.BlockSpec((d, nc), lambda bi, ti: (0, 0)),
        ],
        out_specs=[
            pl.BlockSpec((1, N_HASHES, tt), lambda bi, ti: (bi, 0, ti)),
            pl.BlockSpec((1, N_HASHES, tt), lambda bi, ti: (bi, 0, ti)),
        ],
        out_shape=[
            jax.ShapeDtypeStruct((b, N_HASHES, t), jnp.int32),
            jax.ShapeDtypeStruct((b, N_HASHES, t), jnp.int32),
        ],
    )(q, k, proj2)


def _attn_body(sq_ref, sk_ref, sv_ref, so_ref, lse_ref, *, nb):
    scale = 1.0 / (sq_ref.shape[-1] ** 0.5)
    for n in range(nb):
        p0 = ((n - 1) % nb) * BUCKET
        c0 = n * BUCKET
        qn = sq_ref[0, c0:c0 + BUCKET, :]
        kk = jnp.concatenate(
            [sk_ref[0, p0:p0 + BUCKET, :], sk_ref[0, c0:c0 + BUCKET, :]], axis=0)
        vv = jnp.concatenate(
            [sv_ref[0, p0:p0 + BUCKET, :], sv_ref[0, c0:c0 + BUCKET, :]], axis=0)
        dots = jax.lax.dot_general(
            qn, kk, (((1,), (1,)), ((), ())),
            preferred_element_type=jnp.float32) * scale
        m = jnp.max(dots, axis=1, keepdims=True)
        p = jnp.exp(dots - m)
        s = jnp.sum(p, axis=1, keepdims=True)
        bo = jnp.dot(p, vv, preferred_element_type=jnp.float32) / s
        so_ref[0, c0:c0 + BUCKET, :] = bo
        lse_ref[0, n, :] = (jnp.log(s) + m)[:, 0]


def _attn_pallas(sq, sk, sv):
    B, t, d = sq.shape
    nb = t // BUCKET
    grid = (B,)
    spec = pl.BlockSpec((1, t, d), lambda i: (i, 0, 0))
    return pl.pallas_call(
        functools.partial(_attn_body, nb=nb),
        grid=grid,
        in_specs=[spec, spec, spec],
        out_specs=[
            pl.BlockSpec((1, t, d), lambda i: (i, 0, 0)),
            pl.BlockSpec((1, nb, BUCKET), lambda i: (i, 0, 0)),
        ],
        out_shape=[
            jax.ShapeDtypeStruct((B, t, d), jnp.float32),
            jax.ShapeDtypeStruct((B, nb, BUCKET), jnp.float32),
        ],
    )(sq, sk, sv)


def kernel(q, k, v, proj):
    b, t, d = q.shape
    proj2 = proj.reshape(d, -1)

    bq, bk = _hash_pallas(q, k, proj2)

    stq = jnp.argsort(bq, axis=-1).astype(jnp.int32)
    stk = jnp.argsort(bk, axis=-1).astype(jnp.int32)
    undo = jnp.argsort(stq, axis=-1).astype(jnp.int32)

    bidx = jnp.arange(b)[:, None, None]
    sq = q[bidx, stq].reshape(b * N_HASHES, t, d)
    sk = k[bidx, stk].reshape(b * N_HASHES, t, d)
    sv = v[bidx, stk].reshape(b * N_HASHES, t, d)

    so, lse = _attn_pallas(sq, sk, sv)
    so = so.reshape(b, N_HASHES, t, d)
    slog = lse.reshape(b, N_HASHES, t)

    o = jnp.take_along_axis(so, undo[..., None], axis=2)
    logits = jnp.take_along_axis(slog, undo, axis=2)
    m = jnp.max(logits, axis=1, keepdims=True)
    w = jnp.exp(logits - m)
    w = w / jnp.sum(w, axis=1, keepdims=True)
    return jnp.sum(o * w[..., None], axis=1)


# speedup vs baseline: 1.1493x; 1.0324x over previous
import functools

import jax
import jax.numpy as jnp
from jax.experimental import pallas as pl

N_HASHES = 8
BUCKET = 64


def _hash_body(q_ref, k_ref, p_ref, bq_ref, bk_ref):
    pq = jnp.dot(q_ref[0], p_ref[...], preferred_element_type=jnp.float32)
    pk = jnp.dot(k_ref[0], p_ref[...], preferred_element_type=jnp.float32)
    big = jnp.int32(1 << 30)
    for src, dst in ((pq, bq_ref), (pk, bk_ref)):
        for r in range(N_HASHES):
            pr = src[:, r * BUCKET:(r + 1) * BUCKET]
            iota = jax.lax.broadcasted_iota(jnp.int32, pr.shape, 1)
            mx = jnp.max(pr, axis=1)
            mn = jnp.min(pr, axis=1)
            amx = jnp.min(jnp.where(pr == mx[:, None], iota, big), axis=1)
            amn = jnp.min(jnp.where(pr == mn[:, None], iota, big), axis=1)
            dst[0, r, :] = jnp.where(mx >= -mn, amx, amn + BUCKET)


def _hash_pallas(q, k, proj2):
    b, t, d = q.shape
    tt = 512
    nc = proj2.shape[1]
    grid = (b, t // tt)
    return pl.pallas_call(
        _hash_body,
        grid=grid,
        in_specs=[
            pl.BlockSpec((1, tt, d), lambda bi, ti: (bi, ti, 0)),
            pl.BlockSpec((1, tt, d), lambda bi, ti: (bi, ti, 0)),
            pl.BlockSpec((d, nc), lambda bi, ti: (0, 0)),
        ],
        out_specs=[
            pl.BlockSpec((1, N_HASHES, tt), lambda bi, ti: (bi, 0, ti)),
            pl.BlockSpec((1, N_HASHES, tt), lambda bi, ti: (bi, 0, ti)),
        ],
        out_shape=[
            jax.ShapeDtypeStruct((b, N_HASHES, t), jnp.int32),
            jax.ShapeDtypeStruct((b, N_HASHES, t), jnp.int32),
        ],
    )(q, k, proj2)


def _attn_body(sq_ref, sk_ref, sv_ref, so_ref, lse_ref, dots_ref, *, nb):
    scale = 1.0 / (sq_ref.shape[-1] ** 0.5)
    dd = (((1,), (1,)), ((), ()))
    for n in range(nb):
        p0 = ((n - 1) % nb) * BUCKET
        c0 = n * BUCKET
        qn = sq_ref[0, c0:c0 + BUCKET, :]
        kp = sk_ref[0, p0:p0 + BUCKET, :]
        kn = sk_ref[0, c0:c0 + BUCKET, :]
        dots_ref[c0:c0 + BUCKET, 0:BUCKET] = jax.lax.dot_general(
            qn, kp, dd, preferred_element_type=jnp.float32) * scale
        dots_ref[c0:c0 + BUCKET, BUCKET:2 * BUCKET] = jax.lax.dot_general(
            qn, kn, dd, preferred_element_type=jnp.float32) * scale
    d3 = dots_ref[...].reshape(nb, BUCKET, 2 * BUCKET)
    m = jnp.max(d3, axis=2, keepdims=True)
    e = jnp.exp(d3 - m)
    s = jnp.sum(e, axis=2, keepdims=True)
    dots_ref[...] = (e / s).reshape(nb * BUCKET, 2 * BUCKET)
    lse_ref[0] = (jnp.log(s) + m)[:, :, 0]
    for n in range(nb):
        p0 = ((n - 1) % nb) * BUCKET
        c0 = n * BUCKET
        pp = dots_ref[c0:c0 + BUCKET, 0:BUCKET]
        pc = dots_ref[c0:c0 + BUCKET, BUCKET:2 * BUCKET]
        vp = sv_ref[0, p0:p0 + BUCKET, :]
        vn = sv_ref[0, c0:c0 + BUCKET, :]
        so_ref[0, c0:c0 + BUCKET, :] = (
            jnp.dot(pp, vp, preferred_element_type=jnp.float32)
            + jnp.dot(pc, vn, preferred_element_type=jnp.float32))


def _attn_pallas(sq, sk, sv):
    B, t, d = sq.shape
    nb = t // BUCKET
    grid = (B,)
    spec = pl.BlockSpec((1, t, d), lambda i: (i, 0, 0))
    from jax.experimental.pallas import tpu as pltpu
    return pl.pallas_call(
        functools.partial(_attn_body, nb=nb),
        grid=grid,
        in_specs=[spec, spec, spec],
        out_specs=[
            pl.BlockSpec((1, t, d), lambda i: (i, 0, 0)),
            pl.BlockSpec((1, nb, BUCKET), lambda i: (i, 0, 0)),
        ],
        out_shape=[
            jax.ShapeDtypeStruct((B, t, d), jnp.float32),
            jax.ShapeDtypeStruct((B, nb, BUCKET), jnp.float32),
        ],
        scratch_shapes=[pltpu.VMEM((t, 2 * BUCKET), jnp.float32)],
    )(sq, sk, sv)


def kernel(q, k, v, proj):
    b, t, d = q.shape
    proj2 = proj.reshape(d, -1)

    bq, bk = _hash_pallas(q, k, proj2)

    stq = jnp.argsort(bq, axis=-1).astype(jnp.int32)
    stk = jnp.argsort(bk, axis=-1).astype(jnp.int32)
    undo = jnp.argsort(stq, axis=-1).astype(jnp.int32)

    bidx = jnp.arange(b)[:, None, None]
    sq = q[bidx, stq].reshape(b * N_HASHES, t, d)
    sk = k[bidx, stk].reshape(b * N_HASHES, t, d)
    sv = v[bidx, stk].reshape(b * N_HASHES, t, d)

    so, lse = _attn_pallas(sq, sk, sv)
    so = so.reshape(b, N_HASHES, t, d)
    slog = lse.reshape(b, N_HASHES, t)

    o = jnp.take_along_axis(so, undo[..., None], axis=2)
    logits = jnp.take_along_axis(slog, undo, axis=2)
    m = jnp.max(logits, axis=1, keepdims=True)
    w = jnp.exp(logits - m)
    w = w / jnp.sum(w, axis=1, keepdims=True)
    return jnp.sum(o * w[..., None], axis=1)


# speedup vs baseline: 8.6976x; 7.5678x over previous
import functools

import jax
import jax.numpy as jnp
from jax import lax
from jax.experimental import pallas as pl
from jax.experimental.pallas import tpu as pltpu
from jax.experimental.pallas import tpu_sc as plsc

N_HASHES = 8
BUCKET = 64


@functools.lru_cache(maxsize=None)
def _make_gather3(n_rows, d, n_out, chunk):
    num_cores, num_subcores = 2, 16
    nw = num_cores * num_subcores
    per_w = n_out // nw
    assert n_out % nw == 0 and per_w % chunk == 0 and chunk % 8 == 0
    mesh = plsc.VectorSubcoreMesh(
        core_axis_name="c", subcore_axis_name="s",
        num_cores=num_cores, num_subcores=num_subcores)

    @functools.partial(
        pl.kernel,
        out_type=[jax.ShapeDtypeStruct((n_out, d), jnp.float32)] * 3,
        mesh=mesh,
        compiler_params=pltpu.CompilerParams(use_tc_tiling_on_sc=False, needs_layout_passes=False),
        scratch_types=[
            pltpu.VMEM((chunk,), jnp.int32),
            pltpu.VMEM((chunk,), jnp.int32),
            pltpu.VMEM((chunk, d), jnp.float32),
            pltpu.VMEM((chunk, d), jnp.float32),
            pltpu.VMEM((chunk, d), jnp.float32),
            pltpu.SemaphoreType.DMA,
            pltpu.SemaphoreType.DMA,
            pltpu.SemaphoreType.DMA,
        ],
    )
    def gather3(qf, kf, vf, idxq_hbm, idxk_hbm, sq_hbm, sk_hbm, sv_hbm,
                idxq_v, idxk_v, rq, rk, rv, sq_sem, sk_sem, sv_sem):
        wid = lax.axis_index("s") * num_cores + lax.axis_index("c")
        w0 = wid * per_w

        def body(ci, _):
            base = w0 + ci * chunk
            pltpu.sync_copy(idxq_hbm.at[pl.ds(base, chunk)], idxq_v)
            pltpu.sync_copy(idxk_hbm.at[pl.ds(base, chunk)], idxk_v)
            cq = pltpu.async_copy(qf.at[idxq_v], rq, sq_sem)
            ck = pltpu.async_copy(kf.at[idxk_v], rk, sk_sem)
            cv = pltpu.async_copy(vf.at[idxk_v], rv, sv_sem)
            cq.wait()
            ck.wait()
            cv.wait()
            pltpu.sync_copy(rq, sq_hbm.at[pl.ds(base, chunk)])
            pltpu.sync_copy(rk, sk_hbm.at[pl.ds(base, chunk)])
            pltpu.sync_copy(rv, sv_hbm.at[pl.ds(base, chunk)])
            return 0

        lax.fori_loop(0, per_w // chunk, body, 0)

    return gather3


@functools.lru_cache(maxsize=None)
def _make_count_sort(n_rows, t, n_bins, rows_per_batch, with_undo):
    num_cores, num_subcores = 2, 16
    nw = num_cores * num_subcores
    per_w = n_rows // nw
    assert n_rows % nw == 0
    nseg = 16
    seg = t // nseg
    mesh = plsc.VectorSubcoreMesh(
        core_axis_name="c", subcore_axis_name="s",
        num_cores=num_cores, num_subcores=num_subcores)
    out_type = [jax.ShapeDtypeStruct((n_rows, t), jnp.int32)]
    if with_undo:
        out_type.append(jax.ShapeDtypeStruct((n_rows, t), jnp.int32))

    @functools.partial(
        pl.kernel,
        out_type=out_type,
        mesh=mesh,
        compiler_params=pltpu.CompilerParams(use_tc_tiling_on_sc=False, needs_layout_passes=False),
        scratch_types=[
            pltpu.VMEM((t,), jnp.int32),
            pltpu.VMEM((t,), jnp.int32),
            pltpu.VMEM((t,), jnp.int32),
            pltpu.VMEM((n_bins * nseg,), jnp.int32),
        ],
    )
    def count_sort(ids_hbm, *rest):
        if with_undo:
            stick_hbm, undo_hbm, ids_v, stick_v, undo_v, hist_v = rest
        else:
            stick_hbm, ids_v, stick_v, undo_v, hist_v = rest
        wid = lax.axis_index("s") * num_cores + lax.axis_index("c")
        lanes = lax.iota(jnp.int32, 16)
        seg_base = lanes * seg
        ones = jnp.ones((16,), jnp.int32)

        def row_body(rr, _):
            row = wid * per_w + rr
            boff = (row // rows_per_batch) * t
            pltpu.sync_copy(ids_hbm.at[row], ids_v)

            def zb(c, _):
                hist_v[pl.ds(c * 16, 16)] = jnp.zeros((16,), jnp.int32)
                return 0
            lax.fori_loop(0, n_bins, zb, 0)

            def p1(j, _):
                a = plsc.load_gather(ids_v, [seg_base + j])
                plsc.addupdate_scatter(hist_v, [a * 16 + lanes], ones)
                return 0
            lax.fori_loop(0, seg, p1, 0)

            def p2(c, base):
                v = hist_v[pl.ds(c * 16, 16)]
                incl = plsc.cumsum(v)
                tot = lax.reduce_sum(v, axes=(0,))
                hist_v[pl.ds(c * 16, 16)] = incl - v + base
                return base + tot
            lax.fori_loop(0, n_bins, p2, jnp.int32(0))

            def p3(j, _):
                eidx = seg_base + j
                a = plsc.load_gather(ids_v, [eidx])
                bidx = a * 16 + lanes
                pos = plsc.load_gather(hist_v, [bidx])
                plsc.store_scatter(hist_v, [bidx], pos + 1)
                plsc.store_scatter(stick_v, [pos], eidx + boff)
                if with_undo:
                    plsc.store_scatter(undo_v, [eidx], pos + row * t)
                return 0
            lax.fori_loop(0, seg, p3, 0)

            pltpu.sync_copy(stick_v, stick_hbm.at[row])
            if with_undo:
                pltpu.sync_copy(undo_v, undo_hbm.at[row])
            return 0

        lax.fori_loop(0, per_w, row_body, 0)

    return count_sort


@functools.lru_cache(maxsize=None)
def _make_gather_o(n_rows, d, chunk):
    num_cores, num_subcores = 2, 16
    nw = num_cores * num_subcores
    per_w = n_rows // nw
    assert n_rows % nw == 0 and per_w % chunk == 0 and chunk % 8 == 0
    mesh = plsc.VectorSubcoreMesh(
        core_axis_name="c", subcore_axis_name="s",
        num_cores=num_cores, num_subcores=num_subcores)

    @functools.partial(
        pl.kernel,
        out_type=[jax.ShapeDtypeStruct((n_rows, d), jnp.float32),
                  jax.ShapeDtypeStruct((n_rows,), jnp.float32)],
        mesh=mesh,
        compiler_params=pltpu.CompilerParams(use_tc_tiling_on_sc=False,
                                             needs_layout_passes=False),
        scratch_types=[
            pltpu.VMEM((chunk,), jnp.int32),
            pltpu.VMEM((chunk, d), jnp.float32),
            pltpu.VMEM((chunk,), jnp.float32),
            pltpu.SemaphoreType.DMA,
            pltpu.SemaphoreType.DMA,
        ],
    )
    def gather_o(so_hbm, slog_hbm, uidx_hbm, o_hbm, lg_hbm,
                 idx_v, rows_v, lg_v, sem_r, sem_l):
        wid = lax.axis_index("s") * num_cores + lax.axis_index("c")
        w0 = wid * per_w

        def body(ci, _):
            base = w0 + ci * chunk
            pltpu.sync_copy(uidx_hbm.at[pl.ds(base, chunk)], idx_v)
            cr = pltpu.async_copy(so_hbm.at[idx_v], rows_v, sem_r)
            cl = pltpu.async_copy(slog_hbm.at[idx_v], lg_v, sem_l)
            cr.wait()
            cl.wait()
            pltpu.sync_copy(rows_v, o_hbm.at[pl.ds(base, chunk)])
            pltpu.sync_copy(lg_v, lg_hbm.at[pl.ds(base, chunk)])
            return 0

        lax.fori_loop(0, per_w // chunk, body, 0)

    return gather_o


def _combine_body(o_ref, lg_ref, out_ref):
    lg = lg_ref[0]
    m = jnp.max(lg, axis=0, keepdims=True)
    w = jnp.exp(lg - m)
    w = w / jnp.sum(w, axis=0, keepdims=True)
    acc = None
    for r in range(N_HASHES):
        term = o_ref[0, r] * w[r][:, None]
        acc = term if acc is None else acc + term
    out_ref[0] = acc


def _combine_pallas(o, lg):
    b, R, t, d = o.shape
    tt = 512
    grid = (b, t // tt)
    return pl.pallas_call(
        _combine_body,
        grid=grid,
        in_specs=[
            pl.BlockSpec((1, R, tt, d), lambda bi, ti: (bi, 0, ti, 0)),
            pl.BlockSpec((1, R, tt), lambda bi, ti: (bi, 0, ti)),
        ],
        out_specs=pl.BlockSpec((1, tt, d), lambda bi, ti: (bi, ti, 0)),
        out_shape=jax.ShapeDtypeStruct((b, t, d), jnp.float32),
    )(o, lg)


def _hash_body(q_ref, k_ref, p_ref, bq_ref, bk_ref):
    pq = jnp.dot(q_ref[0], p_ref[...], preferred_element_type=jnp.float32)
    pk = jnp.dot(k_ref[0], p_ref[...], preferred_element_type=jnp.float32)
    for src, dst in ((pq, bq_ref), (pk, bk_ref)):
        for r in range(N_HASHES):
            pr = src[:, r * BUCKET:(r + 1) * BUCKET]
            mx = jnp.max(pr, axis=1)
            mn = jnp.min(pr, axis=1)
            cond = mx >= -mn
            sel = jnp.where(cond[:, None], pr, -pr)
            am = jnp.argmax(sel, axis=1).astype(jnp.int32)
            dst[0, r, :] = jnp.where(cond, am, am + BUCKET)


def _hash_pallas(q, k, proj2):
    b, t, d = q.shape
    tt = 512
    nc = proj2.shape[1]
    grid = (b, t // tt)
    return pl.pallas_call(
        _hash_body,
        grid=grid,
        in_specs=[
            pl.BlockSpec((1, tt, d), lambda bi, ti: (bi, ti, 0)),
            pl.BlockSpec((1, tt, d), lambda bi, ti: (bi, ti, 0)),
            pl.BlockSpec((d, nc), lambda bi, ti: (0, 0)),
        ],
        out_specs=[
            pl.BlockSpec((1, N_HASHES, tt), lambda bi, ti: (bi, 0, ti)),
            pl.BlockSpec((1, N_HASHES, tt), lambda bi, ti: (bi, 0, ti)),
        ],
        out_shape=[
            jax.ShapeDtypeStruct((b, N_HASHES, t), jnp.int32),
            jax.ShapeDtypeStruct((b, N_HASHES, t), jnp.int32),
        ],
    )(q, k, proj2)


def _attn_body(sq_ref, sk_ref, sv_ref, so_ref, lse_ref, dots_ref, *, nb):
    scale = 1.0 / (sq_ref.shape[-1] ** 0.5)
    dd = (((1,), (1,)), ((), ()))
    for n in range(nb):
        p0 = ((n - 1) % nb) * BUCKET
        c0 = n * BUCKET
        qn = sq_ref[0, c0:c0 + BUCKET, :]
        kp = sk_ref[0, p0:p0 + BUCKET, :]
        kn = sk_ref[0, c0:c0 + BUCKET, :]
        dots_ref[c0:c0 + BUCKET, 0:BUCKET] = jax.lax.dot_general(
            qn, kp, dd, preferred_element_type=jnp.float32) * scale
        dots_ref[c0:c0 + BUCKET, BUCKET:2 * BUCKET] = jax.lax.dot_general(
            qn, kn, dd, preferred_element_type=jnp.float32) * scale
    d3 = dots_ref[...].reshape(nb, BUCKET, 2 * BUCKET)
    m = jnp.max(d3, axis=2, keepdims=True)
    e = jnp.exp(d3 - m)
    s = jnp.sum(e, axis=2, keepdims=True)
    dots_ref[...] = (e / s).reshape(nb * BUCKET, 2 * BUCKET)
    lse_ref[0] = (jnp.log(s) + m)[:, :, 0]
    for n in range(nb):
        p0 = ((n - 1) % nb) * BUCKET
        c0 = n * BUCKET
        pp = dots_ref[c0:c0 + BUCKET, 0:BUCKET]
        pc = dots_ref[c0:c0 + BUCKET, BUCKET:2 * BUCKET]
        vp = sv_ref[0, p0:p0 + BUCKET, :]
        vn = sv_ref[0, c0:c0 + BUCKET, :]
        so_ref[0, c0:c0 + BUCKET, :] = (
            jnp.dot(pp, vp, preferred_element_type=jnp.float32)
            + jnp.dot(pc, vn, preferred_element_type=jnp.float32))


def _attn_pallas(sq, sk, sv):
    B, t, d = sq.shape
    nb = t // BUCKET
    grid = (B,)
    spec = pl.BlockSpec((1, t, d), lambda i: (i, 0, 0))
    from jax.experimental.pallas import tpu as pltpu
    return pl.pallas_call(
        functools.partial(_attn_body, nb=nb),
        grid=grid,
        in_specs=[spec, spec, spec],
        out_specs=[
            pl.BlockSpec((1, t, d), lambda i: (i, 0, 0)),
            pl.BlockSpec((1, nb, BUCKET), lambda i: (i, 0, 0)),
        ],
        out_shape=[
            jax.ShapeDtypeStruct((B, t, d), jnp.float32),
            jax.ShapeDtypeStruct((B, nb, BUCKET), jnp.float32),
        ],
        scratch_shapes=[pltpu.VMEM((t, 2 * BUCKET), jnp.float32)],
    )(sq, sk, sv)


def kernel(q, k, v, proj):
    b, t, d = q.shape
    proj2 = proj.reshape(d, -1)

    bq, bk = _hash_pallas(q, k, proj2)

    nb2 = 2 * (t // BUCKET)
    sort_q = _make_count_sort(b * N_HASHES, t, nb2, N_HASHES, True)
    sort_k = _make_count_sort(b * N_HASHES, t, nb2, N_HASHES, False)
    idxq, undo = sort_q(bq.reshape(b * N_HASHES, t))
    (idxk,) = sort_k(bk.reshape(b * N_HASHES, t))
    undo = undo.reshape(b, N_HASHES, t)

    idxq = idxq.reshape(b * N_HASHES * t)
    idxk = idxk.reshape(b * N_HASHES * t)
    gather3 = _make_gather3(b * t, d, b * N_HASHES * t, 512)
    sq, sk, sv = gather3(q.reshape(b * t, d), k.reshape(b * t, d),
                         v.reshape(b * t, d), idxq, idxk)
    sq = sq.reshape(b * N_HASHES, t, d)
    sk = sk.reshape(b * N_HASHES, t, d)
    sv = sv.reshape(b * N_HASHES, t, d)

    so, lse = _attn_pallas(sq, sk, sv)

    go = _make_gather_o(b * N_HASHES * t, d, 512)
    o, lg = go(so.reshape(b * N_HASHES * t, d),
               lse.reshape(b * N_HASHES * t),
               undo.reshape(b * N_HASHES * t))
    return _combine_pallas(o.reshape(b, N_HASHES, t, d),
                           lg.reshape(b, N_HASHES, t))
